# baseline (speedup 1.0000x reference)
import functools

import jax
import jax.numpy as jnp
from jax import lax
from jax.experimental import pallas as pl
from jax.experimental.pallas import tpu as pltpu
from jax.experimental.pallas import tpu_sc as plsc

def _bf16r(x):
    u = plsc.bitcast(x, jnp.int32)
    r = u + 0x7FFF + ((u >> 16) & 1)
    return plsc.bitcast(r & jnp.int32(-65536), jnp.float32)


RADIUS = 0.05
HMIN = -0.02
HMAX = 0.04
NS = 32
CF = 256
CO = 256
EPS = 1e-5
NTEC = 32


def _f1_body(feat_ref, w_ref, o_ref):
    o_ref[0] = lax.dot_general(
        feat_ref[0], w_ref[...], (((0,), (0,)), ((), ())),
        preferred_element_type=jnp.float32)


def _stage_a(features, w1ft, interpret=False):
    B = features.shape[0]
    return pl.pallas_call(
        _f1_body,
        grid=(B,),
        in_specs=[
            pl.BlockSpec((1, CF, 1024), lambda b: (b, 0, 0)),
            pl.BlockSpec((CF, CO), lambda b: (0, 0)),
        ],
        out_specs=pl.BlockSpec((1, 1024, CO), lambda b: (b, 0, 0)),
        out_shape=jax.ShapeDtypeStruct((B, 1024, CO), jnp.float32),
        interpret=interpret,
    )(features, w1ft)


def _make_query(B, N):
    CPT = B * N // NTEC
    TPB = N // CPT
    mesh = plsc.VectorSubcoreMesh(core_axis_name="c", subcore_axis_name="s")

    @functools.partial(
        pl.kernel, mesh=mesh,
        compiler_params=pltpu.CompilerParams(needs_layout_passes=False),
        out_type=[
            jax.ShapeDtypeStruct((B * N * NS, CF), jnp.float32),
            jax.ShapeDtypeStruct((B * N, NS * 8), jnp.float32),
            jax.ShapeDtypeStruct((NTEC, 4, N), jnp.float32),
            jax.ShapeDtypeStruct((NTEC, 8, 16), jnp.float32),
        ],
        scratch_types=[
            pltpu.VMEM((3, N), jnp.float32),
            pltpu.VMEM((CPT, 16), jnp.float32),
            pltpu.VMEM((64,), jnp.int32),
            pltpu.VMEM((NS,), jnp.int32),
            pltpu.VMEM((NS, CF), jnp.float32),
            pltpu.VMEM((NS * 8,), jnp.float32),
            pltpu.VMEM((4, N), jnp.float32),
            pltpu.VMEM((8, 16), jnp.float32),
            pltpu.SemaphoreType.DMA,
        ],
    )
    def q(xyzt_hbm, rotp_hbm, f1_hbm, yf_hbm, rr_hbm, cnt_hbm, m2_hbm,
          xyz_v, rot_v, idxb, gidx, rows_v, rr_v, hist_v, m2_v, sem):
        wid = lax.axis_index("s") * 2 + lax.axis_index("c")
        b = wid // TPB
        i0 = (wid % TPB) * CPT
        pltpu.sync_copy(xyzt_hbm.at[b], xyz_v)
        pltpu.sync_copy(rotp_hbm.at[b, pl.ds(i0, CPT)], rot_v)

        iota = lax.iota(jnp.int32, 16)
        zf = jnp.zeros((16,), jnp.float32)

        def _zh(k, _):
            z = jnp.zeros((16,), jnp.float32)
            hist_v[0, pl.ds(k * 16, 16)] = z
            hist_v[1, pl.ds(k * 16, 16)] = z
            hist_v[2, pl.ds(k * 16, 16)] = z
            hist_v[3, pl.ds(k * 16, 16)] = z
            return 0
        lax.fori_loop(0, N // 16, _zh, 0)

        def _zr(k, _):
            z = jnp.zeros((16,), jnp.float32)
            rr_v[pl.ds(k * 16, 16)] = z
            return 0
        lax.fori_loop(0, NS * 8 // 16, _zr, 0)
        for k in range(8):
            m2_v[k] = zf

        x0 = xyz_v[0, pl.ds(0, 16)][0]
        y0 = xyz_v[1, pl.ds(0, 16)][0]
        z0 = xyz_v[2, pl.ds(0, 16)][0]

        def per_center(ic, carry):
            (padn, wpx, wpy, wpz) = carry
            gc = wid * CPT + ic
            rv = rot_v[ic]
            rvb = _bf16r(rv)
            r0 = rvb[0]
            r1 = rvb[1]
            r2 = rvb[2]
            r3 = rvb[3]
            r4 = rvb[4]
            r5 = rvb[5]
            r6 = rvb[6]
            r7 = rvb[7]
            r8 = rvb[8]
            cx = rv[12]
            cy = rv[13]
            cz = rv[14]
            th = rv[2]

            zi_l = jnp.zeros((16,), jnp.int32)
            idxb[pl.ds(0, 16)] = zi_l
            idxb[pl.ds(16, 16)] = zi_l
            idxb[pl.ds(32, 16)] = zi_l
            idxb[pl.ds(48, 16)] = zi_l

            def chunk(cb, cnt):
                io = lax.iota(jnp.int32, 16)
                px = xyz_v[0, pl.ds(cb * 16, 16)]
                py = xyz_v[1, pl.ds(cb * 16, 16)]
                pz = xyz_v[2, pl.ds(cb * 16, 16)]
                ax = _bf16r(px - cx)
                ay = _bf16r(py - cy)
                az = _bf16r(pz - cz)
                rx = ax * r0 + ay * r3 + az * r6
                ry = ax * r1 + ay * r4 + az * r7
                rz = ax * r2 + ay * r5 + az * r8
                m = ((ry * ry + rz * rz) < th) & (rx > HMIN) & (rx < HMAX)
                cum = plsc.cumsum(m.astype(jnp.int32))
                dest = cum + (cnt - 1)
                plsc.store_scatter(idxb, [dest], io + cb * 16,
                                   mask=m & (dest < NS))
                return cnt + cum[15]

            cnt = lax.fori_loop(0, N // 16, chunk, jnp.int32(0))
            vn = jnp.minimum(cnt, NS)

            a0x = x0 - cx
            a0y = y0 - cy
            a0z = z0 - cz
            p0x = a0x * r0 + a0y * r3 + a0z * r6
            p0y = a0x * r1 + a0y * r4 + a0z * r7
            p0z = a0x * r2 + a0y * r5 + a0z * r8
            npadf = (NS - vn).astype(jnp.float32)
            padn = padn + npadf
            wpx = wpx + npadf * p0x
            wpy = wpy + npadf * p0y
            wpz = wpz + npadf * p0z

            io_c = lax.iota(jnp.int32, 16)
            onesf_l = jnp.ones((16,), jnp.float32)
            zi_c = jnp.zeros((16,), jnp.int32)
            for h in range(2):
                li = idxb[pl.ds(h * 16, 16)]
                lanes = io_c + h * 16
                vmask = lanes < vn
                gx = plsc.load_gather(xyz_v, [zi_c, li])
                gy = plsc.load_gather(xyz_v, [zi_c + 1, li])
                gz = plsc.load_gather(xyz_v, [zi_c + 2, li])
                ax = gx - cx
                ay = gy - cy
                az = gz - cz
                rrx = ax * r0 + ay * r3 + az * r6
                rry = ax * r1 + ay * r4 + az * r7
                rrz = ax * r2 + ay * r5 + az * r8
                plsc.store_scatter(rr_v, [lanes * 8], rrx)
                plsc.store_scatter(rr_v, [lanes * 8 + 1], rry)
                plsc.store_scatter(rr_v, [lanes * 8 + 2], rrz)
                plsc.addupdate_scatter(hist_v, [zi_c, li], onesf_l, mask=vmask)
                plsc.addupdate_scatter(hist_v, [zi_c + 1, li], rrx, mask=vmask)
                plsc.addupdate_scatter(hist_v, [zi_c + 2, li], rry, mask=vmask)
                plsc.addupdate_scatter(hist_v, [zi_c + 3, li], rrz, mask=vmask)
                m2_v[0] = m2_v[0] + rrx * rrx
                m2_v[1] = m2_v[1] + rry * rry
                m2_v[2] = m2_v[2] + rrz * rrz
                m2_v[3] = m2_v[3] + rrx * rry
                m2_v[4] = m2_v[4] + rrx * rrz
                m2_v[5] = m2_v[5] + rry * rrz
                gidx[pl.ds(h * 16, 16)] = li + b * N

            pltpu.async_copy(f1_hbm.at[gidx], rows_v, sem).wait()
            pltpu.sync_copy(rows_v, yf_hbm.at[pl.ds(gc * NS, NS)])
            pltpu.sync_copy(rr_v, rr_hbm.at[gc])
            return (padn, wpx, wpy, wpz)

        init = (jnp.float32(0.0), jnp.float32(0.0), jnp.float32(0.0),
                jnp.float32(0.0))
        (padn, wpx, wpy, wpz) = lax.fori_loop(0, CPT, per_center, init)

        lane0 = iota == 0
        for r, s in ((0, padn), (1, wpx), (2, wpy), (3, wpz)):
            cur = hist_v[r, pl.ds(0, 16)]
            hist_v[r, pl.ds(0, 16)] = cur + jnp.where(lane0, jnp.full((16,), s), zf)
        pltpu.sync_copy(hist_v, cnt_hbm.at[wid])
        pltpu.sync_copy(m2_v, m2_hbm.at[wid])

    return q


def _p1_body(cntw_ref, f1_ref, m2_ref, wx8_ref, g1b1_ref, o_ref, acc):
    b = pl.program_id(0)
    nb = pl.num_programs(0)

    @pl.when(b == 0)
    def _():
        acc[...] = jnp.zeros_like(acc)

    rows4 = jnp.sum(cntw_ref[...], axis=0)
    f1 = f1_ref[0]
    g = lax.dot_general(rows4, f1, (((1,), (0,)), ((), ())),
                        preferred_element_type=jnp.float32)
    s = lax.dot_general(rows4[0:1], f1 * f1, (((1,), (0,)), ((), ())),
                        preferred_element_type=jnp.float32)
    w = jnp.sum(rows4[1:4], axis=1, keepdims=True)
    acc[0:4] += g
    acc[4:5] += s
    acc[5:8] += jnp.broadcast_to(w, (3, CO))

    @pl.when(b == nb - 1)
    def _():
        npos = jnp.float32(nb * f1_ref.shape[1] * NS)
        m2s = jnp.sum(jnp.sum(m2_ref[...], axis=0), axis=1)
        wxr = wx8_ref[0:3]
        sum1 = acc[0:1] + jnp.sum(wxr * acc[5:8], axis=0, keepdims=True)
        cross = jnp.sum(wxr * acc[1:4], axis=0, keepdims=True)
        quad = (m2s[0] * wxr[0:1] * wxr[0:1]
                + m2s[1] * wxr[1:2] * wxr[1:2]
                + m2s[2] * wxr[2:3] * wxr[2:3]
                + 2.0 * m2s[3] * wxr[0:1] * wxr[1:2]
                + 2.0 * m2s[4] * wxr[0:1] * wxr[2:3]
                + 2.0 * m2s[5] * wxr[1:2] * wxr[2:3])
        sumsq = acc[4:5] + 2.0 * cross + quad
        mean = sum1 / npos
        var = sumsq / npos - mean * mean
        a1 = g1b1_ref[0:1] * lax.rsqrt(var + EPS)
        o_ref[0:1] = a1
        o_ref[1:2] = g1b1_ref[1:2] - mean * a1


def _stage_p1(cntw, f1, m2, wx8, g1b1, interpret=False):
    B, N = f1.shape[0], f1.shape[1]
    tpb = NTEC // B
    return pl.pallas_call(
        _p1_body,
        grid=(B,),
        in_specs=[
            pl.BlockSpec((tpb, 4, N), lambda b: (b, 0, 0)),
            pl.BlockSpec((1, N, CO), lambda b: (b, 0, 0)),
            pl.BlockSpec((NTEC, 8, 16), lambda b: (0, 0, 0)),
            pl.BlockSpec((8, CO), lambda b: (0, 0)),
            pl.BlockSpec((2, CO), lambda b: (0, 0)),
        ],
        out_specs=pl.BlockSpec((2, CO), lambda b: (0, 0)),
        out_shape=jax.ShapeDtypeStruct((2, CO), jnp.float32),
        scratch_shapes=[pltpu.VMEM((8, CO), jnp.float32)],
        interpret=interpret,
    )(cntw, f1, m2, wx8, g1b1)


_TP = 256


def _p2_body(yf_ref, rr_ref, wx8_ref, a1c1_ref, w2_ref, mx_ref, s2_ref, acc):
    t = pl.program_id(0)
    nt = pl.num_programs(0)

    @pl.when(t == 0)
    def _():
        acc[...] = jnp.zeros_like(acc)

    xyzt = lax.dot_general(rr_ref[...], wx8_ref[...], (((1,), (0,)), ((), ())),
                           preferred_element_type=jnp.float32)
    y1 = yf_ref[...] + xyzt
    h = jnp.maximum(y1 * a1c1_ref[0:1] + a1c1_ref[1:2], 0.0)
    y2 = lax.dot_general(h, w2_ref[...], (((1,), (1,)), ((), ())),
                         preferred_element_type=jnp.float32)
    acc[0:1] += jnp.sum(y2, axis=0, keepdims=True)
    acc[1:2] += jnp.sum(y2 * y2, axis=0, keepdims=True)
    mx_ref[...] = jnp.max(y2.reshape(_TP // NS, NS, CO), axis=1)

    @pl.when(t == nt - 1)
    def _():
        s2_ref[...] = acc[...]


def _stage_p2(yf, rr8, wx8, a1c1, w2, interpret=False):
    npos = yf.shape[0]
    nt = npos // _TP
    return pl.pallas_call(
        _p2_body,
        grid=(nt,),
        in_specs=[
            pl.BlockSpec((_TP, CF), lambda t: (t, 0)),
            pl.BlockSpec((_TP, 8), lambda t: (t, 0)),
            pl.BlockSpec((8, CO), lambda t: (0, 0)),
            pl.BlockSpec((2, CO), lambda t: (0, 0)),
            pl.BlockSpec((CO, CF), lambda t: (0, 0)),
        ],
        out_specs=[
            pl.BlockSpec((_TP // NS, CO), lambda t: (t, 0)),
            pl.BlockSpec((2, CO), lambda t: (0, 0)),
        ],
        out_shape=[
            jax.ShapeDtypeStruct((npos // NS, CO), jnp.float32),
            jax.ShapeDtypeStruct((2, CO), jnp.float32),
        ],
        scratch_shapes=[pltpu.VMEM((2, CO), jnp.float32)],
        interpret=interpret,
    )(yf, rr8, wx8, a1c1, w2)


def _p3_body(mx_ref, s2_ref, g2b2_ref, o_ref):
    npos = jnp.float32(pl.num_programs(0) * mx_ref.shape[0] * NS)
    s = s2_ref[...]
    mean = s[0:1] / npos
    var = s[1:2] / npos - mean * mean
    a2 = g2b2_ref[0:1] * lax.rsqrt(var + EPS)
    c2 = g2b2_ref[1:2] - mean * a2
    y = jnp.maximum(mx_ref[...] * a2 + c2, 0.0)
    o_ref[0] = y.T


def _stage_p3(mx, s2, g2b2, B, N, interpret=False):
    nt = mx.shape[0] // _TP
    tb = nt // B
    return pl.pallas_call(
        _p3_body,
        grid=(nt,),
        in_specs=[
            pl.BlockSpec((_TP, CO), lambda t: (t, 0)),
            pl.BlockSpec((2, CO), lambda t: (0, 0)),
            pl.BlockSpec((2, CO), lambda t: (0, 0)),
        ],
        out_specs=pl.BlockSpec((1, CO, _TP), lambda t: (t // tb, 0, t % tb)),
        out_shape=jax.ShapeDtypeStruct((B, CO, N), jnp.float32),
        interpret=interpret,
    )(mx, s2, g2b2)


def kernel(seed_xyz_graspable, seed_features_graspable, vp_rot,
           W1, g1, b1, W2, g2, b2):
    xyz = seed_xyz_graspable
    B, N, _ = xyz.shape
    rot9 = vp_rot.reshape(B, N, 9)
    rotp = jnp.concatenate(
        [rot9, jnp.zeros((B, N, 3), jnp.float32), xyz,
         jnp.zeros((B, N, 1), jnp.float32)], axis=-1)
    xyzt = jnp.transpose(xyz, (0, 2, 1))
    w1ft = jnp.transpose(W1[:, 3:])
    wx8 = jnp.concatenate(
        [jnp.transpose(W1[:, :3]) / RADIUS, jnp.zeros((5, CO), jnp.float32)],
        axis=0)

    f1 = _stage_a(seed_features_graspable, w1ft)
    yf, rr, cntw, m2 = _make_query(B, N)(xyzt, rotp, f1.reshape(B * N, CO))
    a1c1 = _stage_p1(cntw, f1, m2, wx8, jnp.stack([g1, b1]))
    mx, s2 = _stage_p2(yf, rr.reshape(B * N * NS, 8), wx8, a1c1, W2)
    return _stage_p3(mx, s2, jnp.stack([g2, b2]), B, N)


# speedup vs baseline: 1.1970x; 1.1970x over previous
import functools

import jax
import jax.numpy as jnp
from jax import lax
from jax.experimental import pallas as pl
from jax.experimental.pallas import tpu as pltpu
from jax.experimental.pallas import tpu_sc as plsc

def _bf16r(x):
    u = plsc.bitcast(x, jnp.int32)
    r = u + 0x7FFF + ((u >> 16) & 1)
    return plsc.bitcast(r & jnp.int32(-65536), jnp.float32)


RADIUS = 0.05
HMIN = -0.02
HMAX = 0.04
NS = 32
CF = 256
CO = 256
EPS = 1e-5
NTEC = 32


def _f1_body(feat_ref, w_ref, o_ref):
    o_ref[0] = lax.dot_general(
        feat_ref[0], w_ref[...], (((0,), (0,)), ((), ())),
        preferred_element_type=jnp.float32)


def _stage_a(features, w1ft, interpret=False):
    B = features.shape[0]
    return pl.pallas_call(
        _f1_body,
        grid=(B,),
        in_specs=[
            pl.BlockSpec((1, CF, 1024), lambda b: (b, 0, 0)),
            pl.BlockSpec((CF, CO), lambda b: (0, 0)),
        ],
        out_specs=pl.BlockSpec((1, 1024, CO), lambda b: (b, 0, 0)),
        out_shape=jax.ShapeDtypeStruct((B, 1024, CO), jnp.float32),
        interpret=interpret,
    )(features, w1ft)


def _make_query(B, N):
    CPT = B * N // NTEC
    TPB = N // CPT
    mesh = plsc.VectorSubcoreMesh(core_axis_name="c", subcore_axis_name="s")

    @functools.partial(
        pl.kernel, mesh=mesh,
        compiler_params=pltpu.CompilerParams(needs_layout_passes=False),
        out_type=[
            jax.ShapeDtypeStruct((B * N * NS, CF), jnp.float32),
            jax.ShapeDtypeStruct((B * N, NS * 8), jnp.float32),
            jax.ShapeDtypeStruct((NTEC, 4, N), jnp.float32),
            jax.ShapeDtypeStruct((NTEC, 8, 16), jnp.float32),
        ],
        scratch_types=[
            pltpu.VMEM((3, N), jnp.float32),
            pltpu.VMEM((CPT, 16), jnp.float32),
            pltpu.VMEM((64,), jnp.int32),
            pltpu.VMEM((NS,), jnp.int32),
            pltpu.VMEM((NS,), jnp.int32),
            pltpu.VMEM((NS, CF), jnp.float32),
            pltpu.VMEM((NS, CF), jnp.float32),
            pltpu.VMEM((NS * 8,), jnp.float32),
            pltpu.VMEM((NS * 8,), jnp.float32),
            pltpu.VMEM((4, N), jnp.float32),
            pltpu.VMEM((8, 16), jnp.float32),
            pltpu.SemaphoreType.DMA,
            pltpu.SemaphoreType.DMA,
            pltpu.SemaphoreType.DMA,
            pltpu.SemaphoreType.DMA,
        ],
    )
    def q(xyzt_hbm, rotp_hbm, f1_hbm, yf_hbm, rr_hbm, cnt_hbm, m2_hbm,
          xyz_v, rot_v, idxb, gidx0, gidx1, rows0, rows1, rr0, rr1,
          hist_v, m2_v, gsem0, gsem1, wsem0, wsem1):
        gidx_s = (gidx0, gidx1)
        rows_s = (rows0, rows1)
        rr_s = (rr0, rr1)
        gsem_s = (gsem0, gsem1)
        wsem_s = (wsem0, wsem1)
        wid = lax.axis_index("s") * 2 + lax.axis_index("c")
        b = wid // TPB
        i0 = (wid % TPB) * CPT
        pltpu.sync_copy(xyzt_hbm.at[b], xyz_v)
        pltpu.sync_copy(rotp_hbm.at[b, pl.ds(i0, CPT)], rot_v)

        iota = lax.iota(jnp.int32, 16)
        zf = jnp.zeros((16,), jnp.float32)

        def _zh(k, _):
            z = jnp.zeros((16,), jnp.float32)
            hist_v[0, pl.ds(k * 16, 16)] = z
            hist_v[1, pl.ds(k * 16, 16)] = z
            hist_v[2, pl.ds(k * 16, 16)] = z
            hist_v[3, pl.ds(k * 16, 16)] = z
            return 0
        lax.fori_loop(0, N // 16, _zh, 0)

        def _zr(k, _):
            z = jnp.zeros((16,), jnp.float32)
            rr0[pl.ds(k * 16, 16)] = z
            rr1[pl.ds(k * 16, 16)] = z
            return 0
        lax.fori_loop(0, NS * 8 // 16, _zr, 0)
        for k in range(8):
            m2_v[k] = zf

        x0 = xyz_v[0, pl.ds(0, 16)][0]
        y0 = xyz_v[1, pl.ds(0, 16)][0]
        z0 = xyz_v[2, pl.ds(0, 16)][0]

        def _wait_writes(s):
            pltpu.make_async_copy(rows_s[s], yf_hbm.at[pl.ds(0, NS)],
                                  wsem_s[s]).wait()
            pltpu.make_async_copy(rr_s[s], rr_hbm.at[0], wsem_s[s]).wait()

        def _finish_other(s, gc_prev):
            o = 1 - s
            pltpu.make_async_copy(f1_hbm.at[gidx_s[o]], rows_s[o],
                                  gsem_s[o]).wait()
            pltpu.async_copy(rows_s[o], yf_hbm.at[pl.ds(gc_prev * NS, NS)],
                             wsem_s[o])
            pltpu.async_copy(rr_s[o], rr_hbm.at[gc_prev], wsem_s[o])

        def _do_center(ic, s, k, carry, first):
            (padn, wpx, wpy, wpz) = carry
            rv = rot_v[ic]
            rvb = _bf16r(rv)
            r0 = rvb[0]
            r1 = rvb[1]
            r2 = rvb[2]
            r3 = rvb[3]
            r4 = rvb[4]
            r5 = rvb[5]
            r6 = rvb[6]
            r7 = rvb[7]
            r8 = rvb[8]
            cx = rv[12]
            cy = rv[13]
            cz = rv[14]
            th = rv[2]

            zi_l = jnp.zeros((16,), jnp.int32)
            idxb[pl.ds(0, 16)] = zi_l
            idxb[pl.ds(16, 16)] = zi_l
            idxb[pl.ds(32, 16)] = zi_l
            idxb[pl.ds(48, 16)] = zi_l

            def chunk4(q4, cnt):
                io = lax.iota(jnp.int32, 16)
                cums = []
                for j in range(4):
                    cb = q4 * 4 + j
                    px = xyz_v[0, pl.ds(cb * 16, 16)]
                    py = xyz_v[1, pl.ds(cb * 16, 16)]
                    pz = xyz_v[2, pl.ds(cb * 16, 16)]
                    ax = _bf16r(px - cx)
                    ay = _bf16r(py - cy)
                    az = _bf16r(pz - cz)
                    rx = ax * r0 + ay * r3 + az * r6
                    ry = ax * r1 + ay * r4 + az * r7
                    rz = ax * r2 + ay * r5 + az * r8
                    m = (((ry * ry + rz * rz) < th) & (rx > HMIN)
                         & (rx < HMAX))
                    cums.append((m, plsc.cumsum(m.astype(jnp.int32))))
                for j in range(4):
                    m, cum = cums[j]
                    dest = cum + (cnt - 1)
                    plsc.store_scatter(idxb, [dest], io + (q4 * 4 + j) * 16,
                                       mask=m & (dest < NS))
                    cnt = cnt + cum[15]
                return cnt

            cnt = lax.fori_loop(0, N // 64, chunk4, jnp.int32(0))
            vn = jnp.minimum(cnt, NS)

            a0x = x0 - cx
            a0y = y0 - cy
            a0z = z0 - cz
            p0x = a0x * r0 + a0y * r3 + a0z * r6
            p0y = a0x * r1 + a0y * r4 + a0z * r7
            p0z = a0x * r2 + a0y * r5 + a0z * r8
            npadf = (NS - vn).astype(jnp.float32)
            padn = padn + npadf
            wpx = wpx + npadf * p0x
            wpy = wpy + npadf * p0y
            wpz = wpz + npadf * p0z

            if first:
                pass
            else:
                @pl.when(k > 0)
                def _():
                    _wait_writes(s)

            io_c = lax.iota(jnp.int32, 16)
            onesf_l = jnp.ones((16,), jnp.float32)
            zi_c = jnp.zeros((16,), jnp.int32)
            rr_v = rr_s[s]
            for h in range(2):
                li = idxb[pl.ds(h * 16, 16)]
                lanes = io_c + h * 16
                vmask = lanes < vn
                gx = plsc.load_gather(xyz_v, [zi_c, li])
                gy = plsc.load_gather(xyz_v, [zi_c + 1, li])
                gz = plsc.load_gather(xyz_v, [zi_c + 2, li])
                ax = gx - cx
                ay = gy - cy
                az = gz - cz
                rrx = ax * r0 + ay * r3 + az * r6
                rry = ax * r1 + ay * r4 + az * r7
                rrz = ax * r2 + ay * r5 + az * r8
                plsc.store_scatter(rr_v, [lanes * 8], rrx)
                plsc.store_scatter(rr_v, [lanes * 8 + 1], rry)
                plsc.store_scatter(rr_v, [lanes * 8 + 2], rrz)
                plsc.addupdate_scatter(hist_v, [zi_c, li], onesf_l, mask=vmask)
                plsc.addupdate_scatter(hist_v, [zi_c + 1, li], rrx, mask=vmask)
                plsc.addupdate_scatter(hist_v, [zi_c + 2, li], rry, mask=vmask)
                plsc.addupdate_scatter(hist_v, [zi_c + 3, li], rrz, mask=vmask)
                m2_v[0] = m2_v[0] + rrx * rrx
                m2_v[1] = m2_v[1] + rry * rry
                m2_v[2] = m2_v[2] + rrz * rrz
                m2_v[3] = m2_v[3] + rrx * rry
                m2_v[4] = m2_v[4] + rrx * rrz
                m2_v[5] = m2_v[5] + rry * rrz
                gidx_s[s][pl.ds(h * 16, 16)] = li + b * N

            pltpu.async_copy(f1_hbm.at[gidx_s[s]], rows_s[s], gsem_s[s])

            gc_prev = wid * CPT + (ic - 1)
            if first:
                pass
            elif s == 0:
                @pl.when(k > 0)
                def _():
                    _finish_other(s, gc_prev)
            else:
                _finish_other(s, gc_prev)
            return (padn, wpx, wpy, wpz)

        def pair(k, carry):
            carry = _do_center(2 * k, 0, k, carry, False)
            carry = _do_center(2 * k + 1, 1, k, carry, False)
            return carry

        init = (jnp.float32(0.0), jnp.float32(0.0), jnp.float32(0.0),
                jnp.float32(0.0))
        (padn, wpx, wpy, wpz) = lax.fori_loop(0, CPT // 2, pair, init)

        _finish_other(0, wid * CPT + CPT - 1)
        _wait_writes(0)
        _wait_writes(1)

        lane0 = iota == 0
        for r, s in ((0, padn), (1, wpx), (2, wpy), (3, wpz)):
            cur = hist_v[r, pl.ds(0, 16)]
            hist_v[r, pl.ds(0, 16)] = cur + jnp.where(lane0, jnp.full((16,), s), zf)
        pltpu.sync_copy(hist_v, cnt_hbm.at[wid])
        pltpu.sync_copy(m2_v, m2_hbm.at[wid])

    return q


def _p1_body(cntw_ref, f1_ref, m2_ref, wx8_ref, g1b1_ref, o_ref, acc):
    b = pl.program_id(0)
    nb = pl.num_programs(0)

    @pl.when(b == 0)
    def _():
        acc[...] = jnp.zeros_like(acc)

    rows4 = jnp.sum(cntw_ref[...], axis=0)
    f1 = f1_ref[0]
    g = lax.dot_general(rows4, f1, (((1,), (0,)), ((), ())),
                        preferred_element_type=jnp.float32)
    s = lax.dot_general(rows4[0:1], f1 * f1, (((1,), (0,)), ((), ())),
                        preferred_element_type=jnp.float32)
    w = jnp.sum(rows4[1:4], axis=1, keepdims=True)
    acc[0:4] += g
    acc[4:5] += s
    acc[5:8] += jnp.broadcast_to(w, (3, CO))

    @pl.when(b == nb - 1)
    def _():
        npos = jnp.float32(nb * f1_ref.shape[1] * NS)
        m2s = jnp.sum(jnp.sum(m2_ref[...], axis=0), axis=1)
        wxr = wx8_ref[0:3]
        sum1 = acc[0:1] + jnp.sum(wxr * acc[5:8], axis=0, keepdims=True)
        cross = jnp.sum(wxr * acc[1:4], axis=0, keepdims=True)
        quad = (m2s[0] * wxr[0:1] * wxr[0:1]
                + m2s[1] * wxr[1:2] * wxr[1:2]
                + m2s[2] * wxr[2:3] * wxr[2:3]
                + 2.0 * m2s[3] * wxr[0:1] * wxr[1:2]
                + 2.0 * m2s[4] * wxr[0:1] * wxr[2:3]
                + 2.0 * m2s[5] * wxr[1:2] * wxr[2:3])
        sumsq = acc[4:5] + 2.0 * cross + quad
        mean = sum1 / npos
        var = sumsq / npos - mean * mean
        a1 = g1b1_ref[0:1] * lax.rsqrt(var + EPS)
        o_ref[0:1] = a1
        o_ref[1:2] = g1b1_ref[1:2] - mean * a1


def _stage_p1(cntw, f1, m2, wx8, g1b1, interpret=False):
    B, N = f1.shape[0], f1.shape[1]
    tpb = NTEC // B
    return pl.pallas_call(
        _p1_body,
        grid=(B,),
        in_specs=[
            pl.BlockSpec((tpb, 4, N), lambda b: (b, 0, 0)),
            pl.BlockSpec((1, N, CO), lambda b: (b, 0, 0)),
            pl.BlockSpec((NTEC, 8, 16), lambda b: (0, 0, 0)),
            pl.BlockSpec((8, CO), lambda b: (0, 0)),
            pl.BlockSpec((2, CO), lambda b: (0, 0)),
        ],
        out_specs=pl.BlockSpec((2, CO), lambda b: (0, 0)),
        out_shape=jax.ShapeDtypeStruct((2, CO), jnp.float32),
        scratch_shapes=[pltpu.VMEM((8, CO), jnp.float32)],
        interpret=interpret,
    )(cntw, f1, m2, wx8, g1b1)


_TP = 256


def _p2_body(yf_ref, rr_ref, wx8_ref, a1c1_ref, w2_ref, mx_ref, s2_ref, acc):
    t = pl.program_id(0)
    nt = pl.num_programs(0)

    @pl.when(t == 0)
    def _():
        acc[...] = jnp.zeros_like(acc)

    xyzt = lax.dot_general(rr_ref[...], wx8_ref[...], (((1,), (0,)), ((), ())),
                           preferred_element_type=jnp.float32)
    y1 = yf_ref[...] + xyzt
    h = jnp.maximum(y1 * a1c1_ref[0:1] + a1c1_ref[1:2], 0.0)
    y2 = lax.dot_general(h.astype(jnp.bfloat16), w2_ref[...],
                         (((1,), (1,)), ((), ())),
                         preferred_element_type=jnp.float32)
    acc[0:1] += jnp.sum(y2, axis=0, keepdims=True)
    acc[1:2] += jnp.sum(y2 * y2, axis=0, keepdims=True)
    mx_ref[...] = jnp.max(y2.reshape(_TP // NS, NS, CO), axis=1)

    @pl.when(t == nt - 1)
    def _():
        s2_ref[...] = acc[...]


def _stage_p2(yf, rr8, wx8, a1c1, w2, interpret=False):
    npos = yf.shape[0]
    nt = npos // _TP
    return pl.pallas_call(
        _p2_body,
        grid=(nt,),
        in_specs=[
            pl.BlockSpec((_TP, CF), lambda t: (t, 0)),
            pl.BlockSpec((_TP, 8), lambda t: (t, 0)),
            pl.BlockSpec((8, CO), lambda t: (0, 0)),
            pl.BlockSpec((2, CO), lambda t: (0, 0)),
            pl.BlockSpec((CO, CF), lambda t: (0, 0)),
        ],
        out_specs=[
            pl.BlockSpec((_TP // NS, CO), lambda t: (t, 0)),
            pl.BlockSpec((2, CO), lambda t: (0, 0)),
        ],
        out_shape=[
            jax.ShapeDtypeStruct((npos // NS, CO), jnp.float32),
            jax.ShapeDtypeStruct((2, CO), jnp.float32),
        ],
        scratch_shapes=[pltpu.VMEM((2, CO), jnp.float32)],
        interpret=interpret,
    )(yf, rr8, wx8, a1c1, w2)


def _p3_body(mx_ref, s2_ref, g2b2_ref, o_ref):
    npos = jnp.float32(pl.num_programs(0) * mx_ref.shape[0] * NS)
    s = s2_ref[...]
    mean = s[0:1] / npos
    var = s[1:2] / npos - mean * mean
    a2 = g2b2_ref[0:1] * lax.rsqrt(var + EPS)
    c2 = g2b2_ref[1:2] - mean * a2
    y = jnp.maximum(mx_ref[...] * a2 + c2, 0.0)
    o_ref[0] = y.T


def _stage_p3(mx, s2, g2b2, B, N, interpret=False):
    nt = mx.shape[0] // _TP
    tb = nt // B
    return pl.pallas_call(
        _p3_body,
        grid=(nt,),
        in_specs=[
            pl.BlockSpec((_TP, CO), lambda t: (t, 0)),
            pl.BlockSpec((2, CO), lambda t: (0, 0)),
            pl.BlockSpec((2, CO), lambda t: (0, 0)),
        ],
        out_specs=pl.BlockSpec((1, CO, _TP), lambda t: (t // tb, 0, t % tb)),
        out_shape=jax.ShapeDtypeStruct((B, CO, N), jnp.float32),
        interpret=interpret,
    )(mx, s2, g2b2)


def kernel(seed_xyz_graspable, seed_features_graspable, vp_rot,
           W1, g1, b1, W2, g2, b2):
    xyz = seed_xyz_graspable
    B, N, _ = xyz.shape
    rot9 = vp_rot.reshape(B, N, 9)
    rotp = jnp.concatenate(
        [rot9, jnp.zeros((B, N, 3), jnp.float32), xyz,
         jnp.zeros((B, N, 1), jnp.float32)], axis=-1)
    xyzt = jnp.transpose(xyz, (0, 2, 1))
    w1ft = jnp.transpose(W1[:, 3:])
    wx8 = jnp.concatenate(
        [jnp.transpose(W1[:, :3]) / RADIUS, jnp.zeros((5, CO), jnp.float32)],
        axis=0)

    f1 = _stage_a(seed_features_graspable, w1ft)
    yf, rr, cntw, m2 = _make_query(B, N)(xyzt, rotp, f1.reshape(B * N, CO))
    a1c1 = _stage_p1(cntw, f1, m2, wx8, jnp.stack([g1, b1]))
    mx, s2 = _stage_p2(yf, rr.reshape(B * N * NS, 8), wx8, a1c1,
                       W2.astype(jnp.bfloat16))
    return _stage_p3(mx, s2, jnp.stack([g2, b2]), B, N)


# speedup vs baseline: 1.1978x; 1.0007x over previous
import functools

import jax
import jax.numpy as jnp
from jax import lax
from jax.experimental import pallas as pl
from jax.experimental.pallas import tpu as pltpu
from jax.experimental.pallas import tpu_sc as plsc

def _bf16r(x):
    u = plsc.bitcast(x, jnp.int32)
    r = u + 0x7FFF + ((u >> 16) & 1)
    return plsc.bitcast(r & jnp.int32(-65536), jnp.float32)


RADIUS = 0.05
HMIN = -0.02
HMAX = 0.04
NS = 32
CF = 256
CO = 256
EPS = 1e-5
NTEC = 32


def _f1_body(feat_ref, w_ref, o_ref):
    o_ref[0] = lax.dot_general(
        feat_ref[0], w_ref[...], (((0,), (0,)), ((), ())),
        preferred_element_type=jnp.float32)


def _stage_a(features, w1ft, interpret=False):
    B = features.shape[0]
    return pl.pallas_call(
        _f1_body,
        grid=(B,),
        in_specs=[
            pl.BlockSpec((1, CF, 1024), lambda b: (b, 0, 0)),
            pl.BlockSpec((CF, CO), lambda b: (0, 0)),
        ],
        out_specs=pl.BlockSpec((1, 1024, CO), lambda b: (b, 0, 0)),
        out_shape=jax.ShapeDtypeStruct((B, 1024, CO), jnp.float32),
        interpret=interpret,
    )(features, w1ft)


def _make_query(B, N):
    CPT = B * N // NTEC
    TPB = N // CPT
    mesh = plsc.VectorSubcoreMesh(core_axis_name="c", subcore_axis_name="s")

    @functools.partial(
        pl.kernel, mesh=mesh,
        compiler_params=pltpu.CompilerParams(needs_layout_passes=False),
        out_type=[
            jax.ShapeDtypeStruct((B * N * NS, CF), jnp.float32),
            jax.ShapeDtypeStruct((B * N, NS * 8), jnp.float32),
            jax.ShapeDtypeStruct((NTEC, 4, N), jnp.float32),
            jax.ShapeDtypeStruct((NTEC, 8, 16), jnp.float32),
        ],
        scratch_types=[
            pltpu.VMEM((3, N), jnp.float32),
            pltpu.VMEM((CPT, 16), jnp.float32),
            pltpu.VMEM((64,), jnp.int32),
            pltpu.VMEM((NS,), jnp.int32),
            pltpu.VMEM((NS,), jnp.int32),
            pltpu.VMEM((NS, CF), jnp.float32),
            pltpu.VMEM((NS, CF), jnp.float32),
            pltpu.VMEM((NS * 8,), jnp.float32),
            pltpu.VMEM((NS * 8,), jnp.float32),
            pltpu.VMEM((4, N), jnp.float32),
            pltpu.VMEM((8, 16), jnp.float32),
            pltpu.SemaphoreType.DMA,
            pltpu.SemaphoreType.DMA,
            pltpu.SemaphoreType.DMA,
            pltpu.SemaphoreType.DMA,
        ],
    )
    def q(xyzt_hbm, rotp_hbm, f1_hbm, yf_hbm, rr_hbm, cnt_hbm, m2_hbm,
          xyz_v, rot_v, idxb, gidx0, gidx1, rows0, rows1, rr0, rr1,
          hist_v, m2_v, gsem0, gsem1, wsem0, wsem1):
        gidx_s = (gidx0, gidx1)
        rows_s = (rows0, rows1)
        rr_s = (rr0, rr1)
        gsem_s = (gsem0, gsem1)
        wsem_s = (wsem0, wsem1)
        wid = lax.axis_index("s") * 2 + lax.axis_index("c")
        b = wid // TPB
        i0 = (wid % TPB) * CPT
        pltpu.sync_copy(xyzt_hbm.at[b], xyz_v)
        pltpu.sync_copy(rotp_hbm.at[b, pl.ds(i0, CPT)], rot_v)

        iota = lax.iota(jnp.int32, 16)
        zf = jnp.zeros((16,), jnp.float32)

        def _zh(k, _):
            z = jnp.zeros((16,), jnp.float32)
            hist_v[0, pl.ds(k * 16, 16)] = z
            hist_v[1, pl.ds(k * 16, 16)] = z
            hist_v[2, pl.ds(k * 16, 16)] = z
            hist_v[3, pl.ds(k * 16, 16)] = z
            return 0
        lax.fori_loop(0, N // 16, _zh, 0)

        def _zr(k, _):
            z = jnp.zeros((16,), jnp.float32)
            rr0[pl.ds(k * 16, 16)] = z
            rr1[pl.ds(k * 16, 16)] = z
            return 0
        lax.fori_loop(0, NS * 8 // 16, _zr, 0)
        for k in range(8):
            m2_v[k] = zf

        x0 = xyz_v[0, pl.ds(0, 16)][0]
        y0 = xyz_v[1, pl.ds(0, 16)][0]
        z0 = xyz_v[2, pl.ds(0, 16)][0]

        def _wait_writes(s):
            pltpu.make_async_copy(rows_s[s], yf_hbm.at[pl.ds(0, NS)],
                                  wsem_s[s]).wait()
            pltpu.make_async_copy(rr_s[s], rr_hbm.at[0], wsem_s[s]).wait()

        def _finish_other(s, gc_prev):
            o = 1 - s
            pltpu.make_async_copy(f1_hbm.at[gidx_s[o]], rows_s[o],
                                  gsem_s[o]).wait()
            pltpu.async_copy(rows_s[o], yf_hbm.at[pl.ds(gc_prev * NS, NS)],
                             wsem_s[o])
            pltpu.async_copy(rr_s[o], rr_hbm.at[gc_prev], wsem_s[o])

        def _do_center(ic, s, k, carry, first):
            (padn, wpx, wpy, wpz) = carry
            rv = rot_v[ic]
            rvb = _bf16r(rv)
            r0 = rvb[0]
            r1 = rvb[1]
            r2 = rvb[2]
            r3 = rvb[3]
            r4 = rvb[4]
            r5 = rvb[5]
            r6 = rvb[6]
            r7 = rvb[7]
            r8 = rvb[8]
            cx = rv[12]
            cy = rv[13]
            cz = rv[14]
            th = rv[2]

            zi_l = jnp.zeros((16,), jnp.int32)
            idxb[pl.ds(0, 16)] = zi_l
            idxb[pl.ds(16, 16)] = zi_l
            idxb[pl.ds(32, 16)] = zi_l
            idxb[pl.ds(48, 16)] = zi_l

            def chunk4(q4, cntv):
                io = lax.iota(jnp.int32, 16)
                cums = []
                for j in range(4):
                    cb = q4 * 4 + j
                    px = xyz_v[0, pl.ds(cb * 16, 16)]
                    py = xyz_v[1, pl.ds(cb * 16, 16)]
                    pz = xyz_v[2, pl.ds(cb * 16, 16)]
                    ax = _bf16r(px - cx)
                    ay = _bf16r(py - cy)
                    az = _bf16r(pz - cz)
                    rx = ax * r0 + ay * r3 + az * r6
                    ry = ax * r1 + ay * r4 + az * r7
                    rz = ax * r2 + ay * r5 + az * r8
                    m = (((ry * ry + rz * rz) < th) & (rx > HMIN)
                         & (rx < HMAX))
                    pc = plsc.all_reduce_population_count(m)
                    cums.append((m, plsc.cumsum(m.astype(jnp.int32)), pc))
                for j in range(4):
                    m, cum, pc = cums[j]
                    dest = cum + (cntv - 1)
                    plsc.store_scatter(idxb, [dest], io + (q4 * 4 + j) * 16,
                                       mask=m & (dest < NS))
                    cntv = cntv + pc
                return cntv

            cntv = lax.fori_loop(0, N // 64, chunk4,
                                 jnp.zeros((16,), jnp.int32))
            cnt = cntv[0]
            vn = jnp.minimum(cnt, NS)

            a0x = x0 - cx
            a0y = y0 - cy
            a0z = z0 - cz
            p0x = a0x * r0 + a0y * r3 + a0z * r6
            p0y = a0x * r1 + a0y * r4 + a0z * r7
            p0z = a0x * r2 + a0y * r5 + a0z * r8
            npadf = (NS - vn).astype(jnp.float32)
            padn = padn + npadf
            wpx = wpx + npadf * p0x
            wpy = wpy + npadf * p0y
            wpz = wpz + npadf * p0z

            if first:
                pass
            else:
                @pl.when(k > 0)
                def _():
                    _wait_writes(s)

            io_c = lax.iota(jnp.int32, 16)
            onesf_l = jnp.ones((16,), jnp.float32)
            zi_c = jnp.zeros((16,), jnp.int32)
            rr_v = rr_s[s]
            for h in range(2):
                li = idxb[pl.ds(h * 16, 16)]
                lanes = io_c + h * 16
                vmask = lanes < vn
                gx = plsc.load_gather(xyz_v, [zi_c, li])
                gy = plsc.load_gather(xyz_v, [zi_c + 1, li])
                gz = plsc.load_gather(xyz_v, [zi_c + 2, li])
                ax = gx - cx
                ay = gy - cy
                az = gz - cz
                rrx = ax * r0 + ay * r3 + az * r6
                rry = ax * r1 + ay * r4 + az * r7
                rrz = ax * r2 + ay * r5 + az * r8
                plsc.store_scatter(rr_v, [lanes * 8], rrx)
                plsc.store_scatter(rr_v, [lanes * 8 + 1], rry)
                plsc.store_scatter(rr_v, [lanes * 8 + 2], rrz)
                plsc.addupdate_scatter(hist_v, [zi_c, li], onesf_l, mask=vmask)
                plsc.addupdate_scatter(hist_v, [zi_c + 1, li], rrx, mask=vmask)
                plsc.addupdate_scatter(hist_v, [zi_c + 2, li], rry, mask=vmask)
                plsc.addupdate_scatter(hist_v, [zi_c + 3, li], rrz, mask=vmask)
                m2_v[0] = m2_v[0] + rrx * rrx
                m2_v[1] = m2_v[1] + rry * rry
                m2_v[2] = m2_v[2] + rrz * rrz
                m2_v[3] = m2_v[3] + rrx * rry
                m2_v[4] = m2_v[4] + rrx * rrz
                m2_v[5] = m2_v[5] + rry * rrz
                gidx_s[s][pl.ds(h * 16, 16)] = li + b * N

            pltpu.async_copy(f1_hbm.at[gidx_s[s]], rows_s[s], gsem_s[s])

            gc_prev = wid * CPT + (ic - 1)
            if first:
                pass
            elif s == 0:
                @pl.when(k > 0)
                def _():
                    _finish_other(s, gc_prev)
            else:
                _finish_other(s, gc_prev)
            return (padn, wpx, wpy, wpz)

        def pair(k, carry):
            carry = _do_center(2 * k, 0, k, carry, False)
            carry = _do_center(2 * k + 1, 1, k, carry, False)
            return carry

        init = (jnp.float32(0.0), jnp.float32(0.0), jnp.float32(0.0),
                jnp.float32(0.0))
        (padn, wpx, wpy, wpz) = lax.fori_loop(0, CPT // 2, pair, init)

        _finish_other(0, wid * CPT + CPT - 1)
        _wait_writes(0)
        _wait_writes(1)

        lane0 = iota == 0
        for r, s in ((0, padn), (1, wpx), (2, wpy), (3, wpz)):
            cur = hist_v[r, pl.ds(0, 16)]
            hist_v[r, pl.ds(0, 16)] = cur + jnp.where(lane0, jnp.full((16,), s), zf)
        pltpu.sync_copy(hist_v, cnt_hbm.at[wid])
        pltpu.sync_copy(m2_v, m2_hbm.at[wid])

    return q


def _p1_body(cntw_ref, f1_ref, m2_ref, wx8_ref, g1b1_ref, o_ref, acc):
    b = pl.program_id(0)
    nb = pl.num_programs(0)

    @pl.when(b == 0)
    def _():
        acc[...] = jnp.zeros_like(acc)

    rows4 = jnp.sum(cntw_ref[...], axis=0)
    f1 = f1_ref[0]
    g = lax.dot_general(rows4, f1, (((1,), (0,)), ((), ())),
                        preferred_element_type=jnp.float32)
    s = lax.dot_general(rows4[0:1], f1 * f1, (((1,), (0,)), ((), ())),
                        preferred_element_type=jnp.float32)
    w = jnp.sum(rows4[1:4], axis=1, keepdims=True)
    acc[0:4] += g
    acc[4:5] += s
    acc[5:8] += jnp.broadcast_to(w, (3, CO))

    @pl.when(b == nb - 1)
    def _():
        npos = jnp.float32(nb * f1_ref.shape[1] * NS)
        m2s = jnp.sum(jnp.sum(m2_ref[...], axis=0), axis=1)
        wxr = wx8_ref[0:3]
        sum1 = acc[0:1] + jnp.sum(wxr * acc[5:8], axis=0, keepdims=True)
        cross = jnp.sum(wxr * acc[1:4], axis=0, keepdims=True)
        quad = (m2s[0] * wxr[0:1] * wxr[0:1]
                + m2s[1] * wxr[1:2] * wxr[1:2]
                + m2s[2] * wxr[2:3] * wxr[2:3]
                + 2.0 * m2s[3] * wxr[0:1] * wxr[1:2]
                + 2.0 * m2s[4] * wxr[0:1] * wxr[2:3]
                + 2.0 * m2s[5] * wxr[1:2] * wxr[2:3])
        sumsq = acc[4:5] + 2.0 * cross + quad
        mean = sum1 / npos
        var = sumsq / npos - mean * mean
        a1 = g1b1_ref[0:1] * lax.rsqrt(var + EPS)
        o_ref[0:1] = a1
        o_ref[1:2] = g1b1_ref[1:2] - mean * a1


def _stage_p1(cntw, f1, m2, wx8, g1b1, interpret=False):
    B, N = f1.shape[0], f1.shape[1]
    tpb = NTEC // B
    return pl.pallas_call(
        _p1_body,
        grid=(B,),
        in_specs=[
            pl.BlockSpec((tpb, 4, N), lambda b: (b, 0, 0)),
            pl.BlockSpec((1, N, CO), lambda b: (b, 0, 0)),
            pl.BlockSpec((NTEC, 8, 16), lambda b: (0, 0, 0)),
            pl.BlockSpec((8, CO), lambda b: (0, 0)),
            pl.BlockSpec((2, CO), lambda b: (0, 0)),
        ],
        out_specs=pl.BlockSpec((2, CO), lambda b: (0, 0)),
        out_shape=jax.ShapeDtypeStruct((2, CO), jnp.float32),
        scratch_shapes=[pltpu.VMEM((8, CO), jnp.float32)],
        interpret=interpret,
    )(cntw, f1, m2, wx8, g1b1)


_TP = 256


def _p2_body(yf_ref, rr_ref, wx8_ref, a1c1_ref, w2_ref, mx_ref, s2_ref, acc):
    t = pl.program_id(0)
    nt = pl.num_programs(0)

    @pl.when(t == 0)
    def _():
        acc[...] = jnp.zeros_like(acc)

    xyzt = lax.dot_general(rr_ref[...], wx8_ref[...], (((1,), (0,)), ((), ())),
                           preferred_element_type=jnp.float32)
    y1 = yf_ref[...] + xyzt
    h = jnp.maximum(y1 * a1c1_ref[0:1] + a1c1_ref[1:2], 0.0)
    y2 = lax.dot_general(h.astype(jnp.bfloat16), w2_ref[...],
                         (((1,), (1,)), ((), ())),
                         preferred_element_type=jnp.float32)
    acc[0:1] += jnp.sum(y2, axis=0, keepdims=True)
    acc[1:2] += jnp.sum(y2 * y2, axis=0, keepdims=True)
    mx_ref[...] = jnp.max(y2.reshape(_TP // NS, NS, CO), axis=1)

    @pl.when(t == nt - 1)
    def _():
        s2_ref[...] = acc[...]


def _stage_p2(yf, rr8, wx8, a1c1, w2, interpret=False):
    npos = yf.shape[0]
    nt = npos // _TP
    return pl.pallas_call(
        _p2_body,
        grid=(nt,),
        in_specs=[
            pl.BlockSpec((_TP, CF), lambda t: (t, 0)),
            pl.BlockSpec((_TP, 8), lambda t: (t, 0)),
            pl.BlockSpec((8, CO), lambda t: (0, 0)),
            pl.BlockSpec((2, CO), lambda t: (0, 0)),
            pl.BlockSpec((CO, CF), lambda t: (0, 0)),
        ],
        out_specs=[
            pl.BlockSpec((_TP // NS, CO), lambda t: (t, 0)),
            pl.BlockSpec((2, CO), lambda t: (0, 0)),
        ],
        out_shape=[
            jax.ShapeDtypeStruct((npos // NS, CO), jnp.float32),
            jax.ShapeDtypeStruct((2, CO), jnp.float32),
        ],
        scratch_shapes=[pltpu.VMEM((2, CO), jnp.float32)],
        interpret=interpret,
    )(yf, rr8, wx8, a1c1, w2)


def _p3_body(mx_ref, s2_ref, g2b2_ref, o_ref):
    npos = jnp.float32(pl.num_programs(0) * mx_ref.shape[0] * NS)
    s = s2_ref[...]
    mean = s[0:1] / npos
    var = s[1:2] / npos - mean * mean
    a2 = g2b2_ref[0:1] * lax.rsqrt(var + EPS)
    c2 = g2b2_ref[1:2] - mean * a2
    y = jnp.maximum(mx_ref[...] * a2 + c2, 0.0)
    o_ref[0] = y.T


def _stage_p3(mx, s2, g2b2, B, N, interpret=False):
    nt = mx.shape[0] // _TP
    tb = nt // B
    return pl.pallas_call(
        _p3_body,
        grid=(nt,),
        in_specs=[
            pl.BlockSpec((_TP, CO), lambda t: (t, 0)),
            pl.BlockSpec((2, CO), lambda t: (0, 0)),
            pl.BlockSpec((2, CO), lambda t: (0, 0)),
        ],
        out_specs=pl.BlockSpec((1, CO, _TP), lambda t: (t // tb, 0, t % tb)),
        out_shape=jax.ShapeDtypeStruct((B, CO, N), jnp.float32),
        interpret=interpret,
    )(mx, s2, g2b2)


def kernel(seed_xyz_graspable, seed_features_graspable, vp_rot,
           W1, g1, b1, W2, g2, b2):
    xyz = seed_xyz_graspable
    B, N, _ = xyz.shape
    rot9 = vp_rot.reshape(B, N, 9)
    rotp = jnp.concatenate(
        [rot9, jnp.zeros((B, N, 3), jnp.float32), xyz,
         jnp.zeros((B, N, 1), jnp.float32)], axis=-1)
    xyzt = jnp.transpose(xyz, (0, 2, 1))
    w1ft = jnp.transpose(W1[:, 3:])
    wx8 = jnp.concatenate(
        [jnp.transpose(W1[:, :3]) / RADIUS, jnp.zeros((5, CO), jnp.float32)],
        axis=0)

    f1 = _stage_a(seed_features_graspable, w1ft)
    yf, rr, cntw, m2 = _make_query(B, N)(xyzt, rotp, f1.reshape(B * N, CO))
    a1c1 = _stage_p1(cntw, f1, m2, wx8, jnp.stack([g1, b1]))
    mx, s2 = _stage_p2(yf, rr.reshape(B * N * NS, 8), wx8, a1c1,
                       W2.astype(jnp.bfloat16))
    return _stage_p3(mx, s2, jnp.stack([g2, b2]), B, N)
